# baseline (speedup 1.0000x reference)
import functools

import jax
import jax.numpy as jnp
from jax import lax
from jax.experimental import pallas as pl
from jax.experimental.pallas import tpu as pltpu
from jax.experimental.pallas import tpu_sc as plsc

VOCAB = 1000000
EMBED = 32
E = 800000
NB = E // 128
NR = 2 * NB
CHUNK = 128
DEPTH = 6
NCH_HI = 391
NCH_LO = 390
L = 16


def _make_kernel():
    mesh = plsc.VectorSubcoreMesh(core_axis_name="c", subcore_axis_name="s")

    @functools.partial(
        pl.kernel,
        mesh=mesh,
        compiler_params=pltpu.CompilerParams(use_tc_tiling_on_sc=False,
                                             needs_layout_passes=False),
        out_type=jax.ShapeDtypeStruct((8, NB, 8, 128), jnp.float32),
        scratch_types=[
            pltpu.VMEM((DEPTH, CHUNK), jnp.int32),
            pltpu.VMEM((DEPTH * CHUNK, EMBED), jnp.float32),
            pltpu.VMEM((2 * EMBED, 129), jnp.float32),
            pltpu.SemaphoreType.DMA,
            pltpu.SemaphoreType.DMA((DEPTH,)),
            pltpu.SemaphoreType.DMA,
        ],
    )
    def gather_kernel(idx_hbm, table_hbm, out_hbm, idx_v, rows_v, tiles_v,
                      sem_i, sem_g, sem_o):
        wid = lax.axis_index("s") * 2 + lax.axis_index("c")
        nch = lax.select(wid < 20, NCH_HI, NCH_LO)
        r0 = lax.select(wid < 20, wid * NCH_HI,
                        20 * NCH_HI + (wid - 20) * NCH_LO)
        lane = lax.broadcasted_iota(jnp.int32, (L,), 0)

        def fire_idx(t):
            r = r0 + t
            pltpu.async_copy(idx_hbm.at[r // 2, r % 2], idx_v.at[t % DEPTH],
                             sem_i)

        def wait_idx(t):
            pltpu.make_async_copy(idx_hbm.at[0, 0], idx_v.at[t % DEPTH],
                                  sem_i).wait()

        def fire_gather(t):
            p = t % DEPTH
            pltpu.async_copy(
                table_hbm.at[idx_v.at[p]],
                rows_v.at[pl.ds(p * CHUNK, CHUNK)], sem_g.at[p])

        def wait_gather(t):
            p = t % DEPTH
            pltpu.make_async_copy(
                table_hbm.at[idx_v.at[p]],
                rows_v.at[pl.ds(p * CHUNK, CHUNK)], sem_g.at[p]).wait()

        def fire_stores(t):
            r = r0 + t
            b = r // 2
            jt0 = 4 * (r % 2)
            for jt in range(4):
                pltpu.async_copy(
                    tiles_v.at[pl.ds(jt * 8, 8), pl.ds(0, 128)],
                    out_hbm.at[jt0 + jt, b], sem_o)

        def wait_stores(t):
            r = r0 + t
            b = r // 2
            jt0 = 4 * (r % 2)
            for jt in range(4):
                pltpu.make_async_copy(
                    tiles_v.at[pl.ds(jt * 8, 8), pl.ds(0, 128)],
                    out_hbm.at[jt0 + jt, b], sem_o).wait()

        def transpose(t):
            row0 = (t % DEPTH) * CHUNK
            rvecs = [lane + 16 * k for k in range(EMBED // L)]
            cvecs = [lane + 16 * k for k in range(EMBED // L)]
            for i in range(CHUNK):
                ivec = jnp.full((L,), i, jnp.int32)
                svec = jnp.full((L,), row0 + i, jnp.int32)
                for k in range(EMBED // L):
                    vals = plsc.load_gather(rows_v, [svec, cvecs[k]])
                    plsc.store_scatter(tiles_v, [rvecs[k], ivec], vals)

        for h in range(DEPTH - 1):
            fire_idx(h)
            wait_idx(h)
            fire_gather(h)
        fire_idx(DEPTH - 1)

        def body(t, carry):
            wait_gather(t)

            def start_next():
                wait_idx(t + DEPTH - 1)
                fire_gather(t + DEPTH - 1)

            pl.when(t + DEPTH - 1 < nch)(start_next)
            pl.when(t + DEPTH < nch)(lambda: fire_idx(t + DEPTH))
            pl.when(t >= 1)(lambda: wait_stores(t - 1))
            transpose(t)
            fire_stores(t)
            return carry

        lax.fori_loop(0, nch, body, 0)
        wait_stores(nch - 1)

    return gather_kernel


_gather = _make_kernel()


def kernel(graph, table):
    idx = graph.reshape(NB, 128, 2).transpose(0, 2, 1)
    out_p = _gather(idx.astype(jnp.int32), table)
    return out_p.transpose(1, 3, 0, 2).reshape(E, 2 * EMBED)


# speedup vs baseline: 1.0190x; 1.0190x over previous
import functools

import jax
import jax.numpy as jnp
from jax import lax
from jax.experimental import pallas as pl
from jax.experimental.pallas import tpu as pltpu
from jax.experimental.pallas import tpu_sc as plsc

VOCAB = 1000000
EMBED = 32
E = 800000
NB = E // 128
NR = 2 * NB
CHUNK = 128
DEPTH = 6
NCH_HI = 391
NCH_LO = 390
L = 16


def _make_kernel():
    mesh = plsc.VectorSubcoreMesh(core_axis_name="c", subcore_axis_name="s")

    @functools.partial(
        pl.kernel,
        mesh=mesh,
        compiler_params=pltpu.CompilerParams(use_tc_tiling_on_sc=False,
                                             needs_layout_passes=False),
        out_type=jax.ShapeDtypeStruct((8, NB, 8, 128), jnp.float32),
        scratch_types=[
            pltpu.VMEM((DEPTH, CHUNK), jnp.int32),
            pltpu.VMEM((DEPTH * CHUNK, EMBED), jnp.float32),
            pltpu.VMEM((2 * EMBED, 129), jnp.float32),
            pltpu.SemaphoreType.DMA,
            pltpu.SemaphoreType.DMA((DEPTH,)),
            pltpu.SemaphoreType.DMA,
        ],
    )
    def gather_kernel(idx_hbm, table_hbm, out_hbm, idx_v, rows_v, tiles_v,
                      sem_i, sem_g, sem_o):
        wid = lax.axis_index("s") * 2 + lax.axis_index("c")
        nch = lax.select(wid < 20, NCH_HI, NCH_LO)
        r0 = lax.select(wid < 20, wid * NCH_HI,
                        20 * NCH_HI + (wid - 20) * NCH_LO)
        lane = lax.broadcasted_iota(jnp.int32, (L,), 0)

        def fire_idx(t):
            r = r0 + t
            pltpu.async_copy(idx_hbm.at[r // 2, r % 2], idx_v.at[t % DEPTH],
                             sem_i)

        def wait_idx(t):
            pltpu.make_async_copy(idx_hbm.at[0, 0], idx_v.at[t % DEPTH],
                                  sem_i).wait()

        def fire_gather(t):
            p = t % DEPTH
            pltpu.async_copy(
                table_hbm.at[idx_v.at[p]],
                rows_v.at[pl.ds(p * CHUNK, CHUNK)], sem_g.at[p])

        def wait_gather(t):
            p = t % DEPTH
            pltpu.make_async_copy(
                table_hbm.at[idx_v.at[p]],
                rows_v.at[pl.ds(p * CHUNK, CHUNK)], sem_g.at[p]).wait()

        def fire_stores(t):
            r = r0 + t
            b = r // 2
            jt0 = 4 * (r % 2)
            for jt in range(4):
                pltpu.async_copy(
                    tiles_v.at[pl.ds(jt * 8, 8), pl.ds(0, 128)],
                    out_hbm.at[jt0 + jt, b], sem_o)

        def wait_stores(t):
            r = r0 + t
            b = r // 2
            jt0 = 4 * (r % 2)
            for jt in range(4):
                pltpu.make_async_copy(
                    tiles_v.at[pl.ds(jt * 8, 8), pl.ds(0, 128)],
                    out_hbm.at[jt0 + jt, b], sem_o).wait()

        def transpose(t):
            row0 = (t % DEPTH) * CHUNK
            rvecs = [lane + 16 * k for k in range(EMBED // L)]
            cvecs = [lane + 16 * k for k in range(EMBED // L)]
            for i in range(CHUNK):
                ivec = jnp.full((L,), i, jnp.int32)
                svec = jnp.full((L,), row0 + i, jnp.int32)
                for k in range(EMBED // L):
                    vals = plsc.load_gather(rows_v, [svec, cvecs[k]])
                    plsc.store_scatter(tiles_v, [rvecs[k], ivec], vals)

        for h in range(DEPTH - 1):
            fire_idx(h)
            wait_idx(h)
            fire_gather(h)
        fire_idx(DEPTH - 1)

        def body(t, carry):
            wait_gather(t)

            def start_next():
                wait_idx(t + DEPTH - 1)
                fire_gather(t + DEPTH - 1)

            pl.when(t + DEPTH - 1 < nch)(start_next)
            pl.when(t + DEPTH < nch)(lambda: fire_idx(t + DEPTH))
            pl.when(t >= 1)(lambda: wait_stores(t - 1))
            transpose(t)
            fire_stores(t)
            return carry

        lax.fori_loop(0, nch, body, 0)
        wait_stores(nch - 1)

    return gather_kernel


_gather = _make_kernel()

VPAD = 1000064
NT = VPAD // 128
NBLK_HI = 245
NBLK_LO = 244


def _make_detile():
    mesh = plsc.VectorSubcoreMesh(core_axis_name="c", subcore_axis_name="s")

    @functools.partial(
        pl.kernel,
        mesh=mesh,
        compiler_params=pltpu.CompilerParams(use_tc_tiling_on_sc=False,
                                             needs_layout_passes=False),
        out_type=jax.ShapeDtypeStruct((VPAD, EMBED), jnp.float32),
        scratch_types=[
            pltpu.VMEM((3 * EMBED, 128), jnp.float32),
            pltpu.VMEM((128, EMBED + 1), jnp.float32),
            pltpu.SemaphoreType.DMA((3,)),
            pltpu.SemaphoreType.DMA,
        ],
    )
    def detile_kernel(t4_hbm, out_hbm, tin, tout, sem_in, sem_out):
        wid = lax.axis_index("s") * 2 + lax.axis_index("c")
        nblk = lax.select(wid < 5, NBLK_HI, NBLK_LO)
        v0 = lax.select(wid < 5, wid * NBLK_HI,
                        5 * NBLK_HI + (wid - 5) * NBLK_LO)
        lane = lax.broadcasted_iota(jnp.int32, (L,), 0)
        ivecs = [lane + 16 * k2 for k2 in range(128 // L)]

        def fire_in(v):
            q = v % 3
            for jt in range(4):
                pltpu.async_copy(t4_hbm.at[jt, v0 + v],
                                 tin.at[pl.ds(q * EMBED + jt * 8, 8)],
                                 sem_in.at[q])

        def wait_in(v):
            q = v % 3
            for jt in range(4):
                pltpu.make_async_copy(t4_hbm.at[jt, v0],
                                      tin.at[pl.ds(q * EMBED + jt * 8, 8)],
                                      sem_in.at[q]).wait()

        def fire_out(v):
            pltpu.async_copy(tout.at[pl.ds(0, 128), pl.ds(0, EMBED)],
                             out_hbm.at[pl.ds((v0 + v) * 128, 128)], sem_out)

        def wait_out(v):
            pltpu.make_async_copy(tout.at[pl.ds(0, 128), pl.ds(0, EMBED)],
                                  out_hbm.at[pl.ds(v0 * 128, 128)],
                                  sem_out).wait()

        fire_in(0)
        fire_in(1)

        def body(v, carry):
            q = v % 3
            wait_in(v)
            pl.when(v + 2 < nblk)(lambda: fire_in(v + 2))
            pl.when(v >= 1)(lambda: wait_out(v - 1))
            for c in range(EMBED):
                svec = jnp.full((L,), q * EMBED + c, jnp.int32)
                cvec = jnp.full((L,), c, jnp.int32)
                for k2 in range(128 // L):
                    vals = plsc.load_gather(tin, [svec, ivecs[k2]])
                    plsc.store_scatter(tout, [ivecs[k2], cvec], vals)
            fire_out(v)
            return carry

        lax.fori_loop(0, nblk, body, 0)
        wait_out(nblk - 1)

    return detile_kernel


_detile = _make_detile()


def kernel(graph, table):
    idx = graph.reshape(NB, 128, 2).transpose(0, 2, 1)
    tp = jnp.pad(table, ((0, VPAD - VOCAB), (0, 0)))
    t4 = tp.T.reshape(4, 8, NT, 128).transpose(0, 2, 1, 3)
    tlin = _detile(t4)
    out_p = _gather(idx.astype(jnp.int32), tlin)
    return out_p.transpose(1, 3, 0, 2).reshape(E, 2 * EMBED)
